# Initial kernel scaffold; baseline (speedup 1.0000x reference)
import jax
import jax.numpy as jnp
from jax.experimental import pallas as pl


def kernel(h, pos, edge_index, W1, b1, W2, b2, W3, b3, W4, b4, W5, b5):
    raise NotImplementedError("write your pallas kernel here")


# speedup vs baseline: 1.7968x; 1.7968x over previous
import jax
import jax.numpy as jnp
from jax import lax
from jax.experimental import pallas as pl
from jax.experimental.pallas import tpu as pltpu
from jax.experimental.pallas import tpu_sc as plsc

F32 = jnp.float32

_N = 10000
_E = 320000
_FEAT = 128
_HID = 256

_TW_G = 144
_TW_E = 272

_NC, _NS = 2, 16
_NW = _NC * _NS

_GC = 80
_CH = 80
_HALF = _N // 2
_ACC_R = _HALF + 8
_RPT = _ACC_R // _NS


def _sc_gather(table, src, dst):
    per_w = _E // _NW
    n_ch = per_w // _GC
    mesh = plsc.VectorSubcoreMesh(core_axis_name="c", subcore_axis_name="s")

    def body(table_hbm, src_hbm, dst_hbm, gs_hbm, gd_hbm, idx_v, rows_v, sem):
        c = lax.axis_index("c")
        s = lax.axis_index("s")
        wid = s * _NC + c
        base = wid * per_w

        def step(i, carry):
            off = base + i * _GC
            pltpu.sync_copy(src_hbm.at[pl.ds(off, _GC)], idx_v)
            pltpu.async_copy(table_hbm.at[idx_v], rows_v, sem).wait()
            pltpu.sync_copy(rows_v, gs_hbm.at[pl.ds(off, _GC)])
            pltpu.sync_copy(dst_hbm.at[pl.ds(off, _GC)], idx_v)
            pltpu.async_copy(table_hbm.at[idx_v], rows_v, sem).wait()
            pltpu.sync_copy(rows_v, gd_hbm.at[pl.ds(off, _GC)])
            return carry

        lax.fori_loop(0, n_ch, step, 0)

    f = pl.kernel(
        body,
        out_type=(
            jax.ShapeDtypeStruct((_E, _TW_G), F32),
            jax.ShapeDtypeStruct((_E, _TW_G), F32),
        ),
        mesh=mesh,
        scratch_types=[
            pltpu.VMEM((_GC,), jnp.int32),
            pltpu.VMEM((_GC, _TW_G), F32),
            pltpu.SemaphoreType.DMA,
        ],
        compiler_params=pltpu.CompilerParams(use_tc_tiling_on_sc=False),
    )
    return f(table, src, dst)


def _sc_scatter(eout, dst):
    per_t = _E // _NS
    n_ch = per_t // _CH
    mesh = plsc.VectorSubcoreMesh(core_axis_name="c", subcore_axis_name="s")

    def body(eout_hbm, dst_hbm, agg_hbm, dstc_v, lidx_v, rows_v, acc_sh, sem):
        c = lax.axis_index("c")
        s = lax.axis_index("s")
        node_base = c * _HALF

        zeros16 = jnp.zeros((16,), F32)

        def zrow(i, carry):
            for j in range(_TW_E // 16):
                rows_v[i, pl.ds(j * 16, 16)] = zeros16
            return carry

        lax.fori_loop(0, _CH, zrow, 0)
        r0 = s * _RPT
        pltpu.sync_copy(rows_v, acc_sh.at[pl.ds(r0, _CH)])
        pltpu.sync_copy(rows_v, acc_sh.at[pl.ds(r0 + _CH, _CH)])
        pltpu.sync_copy(rows_v, acc_sh.at[pl.ds(r0 + 2 * _CH, _CH)])
        pltpu.sync_copy(rows_v.at[pl.ds(0, _RPT - 3 * _CH)],
                        acc_sh.at[pl.ds(r0 + 3 * _CH, _RPT - 3 * _CH)])
        plsc.subcore_barrier()

        def step(i, carry):
            off = s * per_t + i * _CH
            pltpu.sync_copy(dst_hbm.at[pl.ds(off, _CH)], dstc_v)
            pltpu.sync_copy(eout_hbm.at[pl.ds(off, _CH)], rows_v)
            for j in range(_CH // 16):
                v = dstc_v[pl.ds(j * 16, 16)]
                lv = v - node_base
                ok = (lv >= 0) & (lv < _HALF)
                lidx_v[pl.ds(j * 16, 16)] = jnp.where(ok, lv, _HALF)
            pltpu.sync_copy(rows_v, acc_sh.at[lidx_v], add=True)
            return carry

        lax.fori_loop(0, n_ch, step, 0)
        plsc.subcore_barrier()

        pltpu.sync_copy(acc_sh.at[pl.ds(r0, _RPT)],
                        agg_hbm.at[c, pl.ds(r0, _RPT)])

    f = pl.kernel(
        body,
        out_type=jax.ShapeDtypeStruct((_NC, _ACC_R, _TW_E), F32),
        mesh=mesh,
        scratch_types=[
            pltpu.VMEM((_CH,), jnp.int32),
            pltpu.VMEM((_CH,), jnp.int32),
            pltpu.VMEM((_CH, _TW_E), F32),
            pltpu.VMEM_SHARED((_ACC_R, _TW_E), F32),
            pltpu.SemaphoreType.DMA,
        ],
        compiler_params=pltpu.CompilerParams(use_tc_tiling_on_sc=False),
    )
    return f(eout, dst)


_BE = 1280


def _edge_mlp(gs, gd, w1a, w1b, w1c, b1, w2, b2, w5, b5):
    def body(gs_ref, gd_ref, w1a_ref, w1b_ref, w1c_ref, b1_ref, w2_ref,
             b2_ref, w5_ref, b5_ref, out_ref):
        gsv = gs_ref[...]
        gdv = gd_ref[...]
        hs = gsv[:, :_FEAT]
        hd = gdv[:, :_FEAT]
        diff = gsv[:, _FEAT:_FEAT + 3] - gdv[:, _FEAT:_FEAT + 3]
        nrm = jnp.sqrt(jnp.sum(diff * diff, axis=-1, keepdims=True))
        dist = nrm + 1e-8
        dist2 = dist * dist
        pre = (jnp.dot(hs, w1a_ref[...], preferred_element_type=F32)
               + jnp.dot(hd, w1b_ref[...], preferred_element_type=F32)
               + dist2 * w1c_ref[...] + b1_ref[...])
        m = pre * jax.nn.sigmoid(pre)
        pre2 = jnp.dot(m, w2_ref[...], preferred_element_type=F32) + b2_ref[...]
        mij = pre2 * jax.nn.sigmoid(pre2)
        wgt = jax.nn.sigmoid(
            jnp.dot(mij, w5_ref[...], preferred_element_type=F32) + b5_ref[...])
        out_ref[:, :_HID] = mij
        out_ref[:, _HID:_HID + 3] = wgt * (diff / dist) * 0.1
        out_ref[:, _HID + 3:] = jnp.zeros((_BE, _TW_E - _HID - 3), F32)

    def wspec(r, c):
        return pl.BlockSpec((r, c), lambda i: (0, 0))

    return pl.pallas_call(
        body,
        grid=(_E // _BE,),
        in_specs=[
            pl.BlockSpec((_BE, _TW_G), lambda i: (i, 0)),
            pl.BlockSpec((_BE, _TW_G), lambda i: (i, 0)),
            wspec(_FEAT, _HID), wspec(_FEAT, _HID), wspec(1, _HID),
            wspec(1, _HID), wspec(_HID, _HID), wspec(1, _HID),
            wspec(_HID, 1), wspec(1, 1),
        ],
        out_specs=pl.BlockSpec((_BE, _TW_E), lambda i: (i, 0)),
        out_shape=jax.ShapeDtypeStruct((_E, _TW_E), F32),
    )(gs, gd, w1a, w1b, w1c, b1, w2, b2, w5, b5)


_BN = 2000


def _node_mlp(h2, pos2, agg, w3a, w3b, b3, w4, b4):
    def body(h_ref, pos_ref, agg_ref, w3a_ref, w3b_ref, b3_ref, w4_ref,
             b4_ref, hn_ref, xn_ref):
        hh = h_ref[...]
        magg = agg_ref[:, :_HID]
        upd = agg_ref[:, _HID:_HID + 3]
        pre = (jnp.dot(hh, w3a_ref[...], preferred_element_type=F32)
               + jnp.dot(magg, w3b_ref[...], preferred_element_type=F32)
               + b3_ref[...])
        nh = pre * jax.nn.sigmoid(pre)
        hn_ref[...] = hh + jnp.dot(nh, w4_ref[...],
                                   preferred_element_type=F32) + b4_ref[...]
        xn_ref[...] = pos_ref[...] + upd

    def wspec(r, c):
        return pl.BlockSpec((r, c), lambda i: (0, 0))

    return pl.pallas_call(
        body,
        grid=(_N // _BN,),
        in_specs=[
            pl.BlockSpec((_BN, _FEAT), lambda i: (i, 0)),
            pl.BlockSpec((_BN, 3), lambda i: (i, 0)),
            pl.BlockSpec((_BN, _TW_E), lambda i: (i, 0)),
            wspec(_FEAT, _HID), wspec(_HID, _HID), wspec(1, _HID),
            wspec(_HID, _FEAT), wspec(1, _FEAT),
        ],
        out_specs=(
            pl.BlockSpec((_BN, _FEAT), lambda i: (i, 0)),
            pl.BlockSpec((_BN, 3), lambda i: (i, 0)),
        ),
        out_shape=(
            jax.ShapeDtypeStruct((_N, _FEAT), F32),
            jax.ShapeDtypeStruct((_N, 3), F32),
        ),
    )(h2, pos2, agg, w3a, w3b, b3, w4, b4)


def kernel(h, pos, edge_index, W1, b1, W2, b2, W3, b3, W4, b4, W5, b5):
    h2 = h[0]
    pos2 = pos[0]
    src = edge_index[0]
    dst = edge_index[1]

    table = jnp.concatenate(
        [h2, pos2, jnp.zeros((_N, _TW_G - _FEAT - 3), F32)], axis=1)
    gs, gd = _sc_gather(table, src, dst)

    eout = _edge_mlp(
        gs, gd,
        W1[:_FEAT], W1[_FEAT:2 * _FEAT], W1[2 * _FEAT:2 * _FEAT + 1],
        b1[None, :], W2, b2[None, :], W5, b5[None, :])

    aggp = _sc_scatter(eout, dst)
    agg = jnp.concatenate([aggp[0, :_HALF], aggp[1, :_HALF]], axis=0)

    h_new, x_new = _node_mlp(
        h2, pos2, agg, W3[:_FEAT], W3[_FEAT:], b3[None, :], W4, b4[None, :])
    return h_new[None], x_new[None]


# speedup vs baseline: 2.2114x; 1.2308x over previous
import jax
import jax.numpy as jnp
from jax import lax
from jax.experimental import pallas as pl
from jax.experimental.pallas import tpu as pltpu
from jax.experimental.pallas import tpu_sc as plsc

F32 = jnp.float32
BF16 = jnp.bfloat16

_N = 10000
_E = 320000
_FEAT = 128
_HID = 256

_TW_G = 160
_TW_E = 256
_CW = 16

_NC, _NS = 2, 16
_NW = _NC * _NS

_GC = 400
_GSUB = 80
_CH = 80
_ACC_R = 10016
_RPT = _ACC_R // _NS


def _sc_gather(table, src, dst):
    per_w = _E // _NW
    n_ch = per_w // _GC
    mesh = plsc.VectorSubcoreMesh(core_axis_name="c", subcore_axis_name="s")

    def body(table_hbm, src_hbm, dst_hbm, gs_hbm, gd_hbm,
             idxs_v, idxd_v, rs_v, rd_v, sem):
        c = lax.axis_index("c")
        s = lax.axis_index("s")
        wid = s * _NC + c
        base = wid * per_w

        def step(i, carry):
            off = base + i * _GC
            pltpu.sync_copy(src_hbm.at[pl.ds(off, _GC)], idxs_v)
            pltpu.sync_copy(dst_hbm.at[pl.ds(off, _GC)], idxd_v)
            descs = []
            for j in range(_GC // _GSUB):
                r = pl.ds(j * _GSUB, _GSUB)
                descs.append(pltpu.async_copy(
                    table_hbm.at[idxs_v.at[r]], rs_v.at[r], sem))
                descs.append(pltpu.async_copy(
                    table_hbm.at[idxd_v.at[r]], rd_v.at[r], sem))
            for d in descs:
                d.wait()
            pltpu.sync_copy(rs_v, gs_hbm.at[pl.ds(off, _GC)])
            pltpu.sync_copy(rd_v, gd_hbm.at[pl.ds(off, _GC)])
            return carry

        lax.fori_loop(0, n_ch, step, 0)

    f = pl.kernel(
        body,
        out_type=(
            jax.ShapeDtypeStruct((_E, _TW_G), BF16),
            jax.ShapeDtypeStruct((_E, _TW_G), BF16),
        ),
        mesh=mesh,
        scratch_types=[
            pltpu.VMEM((_GC,), jnp.int32),
            pltpu.VMEM((_GC,), jnp.int32),
            pltpu.VMEM((_GC, _TW_G), BF16),
            pltpu.VMEM((_GC, _TW_G), BF16),
            pltpu.SemaphoreType.DMA,
        ],
        compiler_params=pltpu.CompilerParams(use_tc_tiling_on_sc=False),
    )
    return f(table, src, dst)


def _sc_scatter(eout, eoutc, dst):
    per_c = _E // _NC
    per_t = per_c // _NS
    n_ch = per_t // _CH
    mesh = plsc.VectorSubcoreMesh(core_axis_name="c", subcore_axis_name="s")

    def body(eout_hbm, eoutc_hbm, dst_hbm, agg_hbm, aggc_hbm,
             dstc_v, rows_v, rowsc_v, acc_sh, accc_sh, sem):
        c = lax.axis_index("c")
        s = lax.axis_index("s")

        zb16 = jnp.zeros((16,), BF16)
        zf16 = jnp.zeros((16,), F32)

        def zrow(i, carry):
            for j in range(_TW_E // 16):
                rows_v[i, pl.ds(j * 16, 16)] = zb16
            rowsc_v[i, pl.ds(0, 16)] = zf16
            return carry

        lax.fori_loop(0, _CH, zrow, 0)
        r0 = s * _RPT
        for k in range(_RPT // _CH):
            pltpu.sync_copy(rows_v, acc_sh.at[pl.ds(r0 + k * _CH, _CH)])
            pltpu.sync_copy(rowsc_v, accc_sh.at[pl.ds(r0 + k * _CH, _CH)])
        rem = _RPT - (_RPT // _CH) * _CH
        pltpu.sync_copy(rows_v.at[pl.ds(0, rem)],
                        acc_sh.at[pl.ds(r0 + (_RPT // _CH) * _CH, rem)])
        pltpu.sync_copy(rowsc_v.at[pl.ds(0, rem)],
                        accc_sh.at[pl.ds(r0 + (_RPT // _CH) * _CH, rem)])
        plsc.subcore_barrier()

        def step(i, carry):
            off = c * per_c + s * per_t + i * _CH
            pltpu.sync_copy(dst_hbm.at[pl.ds(off, _CH)], dstc_v)
            pltpu.sync_copy(eout_hbm.at[pl.ds(off, _CH)], rows_v)
            pltpu.sync_copy(eoutc_hbm.at[pl.ds(off, _CH)], rowsc_v)
            pltpu.sync_copy(rows_v, acc_sh.at[dstc_v], add=True)
            pltpu.sync_copy(rowsc_v, accc_sh.at[dstc_v], add=True)
            return carry

        lax.fori_loop(0, n_ch, step, 0)
        plsc.subcore_barrier()

        pltpu.sync_copy(acc_sh.at[pl.ds(r0, _RPT)],
                        agg_hbm.at[c, pl.ds(r0, _RPT)])
        pltpu.sync_copy(accc_sh.at[pl.ds(r0, _RPT)],
                        aggc_hbm.at[c, pl.ds(r0, _RPT)])

    f = pl.kernel(
        body,
        out_type=(
            jax.ShapeDtypeStruct((_NC, _ACC_R, _TW_E), BF16),
            jax.ShapeDtypeStruct((_NC, _ACC_R, _CW), F32),
        ),
        mesh=mesh,
        scratch_types=[
            pltpu.VMEM((_CH,), jnp.int32),
            pltpu.VMEM((_CH, _TW_E), BF16),
            pltpu.VMEM((_CH, _CW), F32),
            pltpu.VMEM_SHARED((_ACC_R, _TW_E), BF16),
            pltpu.VMEM_SHARED((_ACC_R, _CW), F32),
            pltpu.SemaphoreType.DMA,
        ],
        compiler_params=pltpu.CompilerParams(use_tc_tiling_on_sc=False),
    )
    return f(eout, eoutc, dst)


_BE = 1280


def _edge_mlp(gs, gd, w1a, w1b, w1c, b1, w2, b2, w5, b5):
    def body(gs_ref, gd_ref, w1a_ref, w1b_ref, w1c_ref, b1_ref, w2_ref,
             b2_ref, w5_ref, b5_ref, out_ref, outc_ref):
        gsv = gs_ref[...]
        gdv = gd_ref[...]
        hs = gsv[:, :_FEAT]
        hd = gdv[:, :_FEAT]
        ps = gsv[:, _FEAT:_FEAT + 3].astype(F32)
        pd = gdv[:, _FEAT:_FEAT + 3].astype(F32)
        diff = ps - pd
        nrm = jnp.sqrt(jnp.sum(diff * diff, axis=-1, keepdims=True))
        dist = nrm + 1e-8
        dist2 = dist * dist
        pre = (jnp.dot(hs, w1a_ref[...], preferred_element_type=F32)
               + jnp.dot(hd, w1b_ref[...], preferred_element_type=F32)
               + dist2 * w1c_ref[...] + b1_ref[...])
        m = (pre * jax.nn.sigmoid(pre)).astype(BF16)
        pre2 = jnp.dot(m, w2_ref[...], preferred_element_type=F32) + b2_ref[...]
        mij = pre2 * jax.nn.sigmoid(pre2)
        wgt = jax.nn.sigmoid(
            jnp.dot(mij.astype(BF16), w5_ref[...],
                    preferred_element_type=F32) + b5_ref[...])
        out_ref[...] = mij.astype(BF16)
        outc_ref[:, :3] = wgt * (diff / dist) * 0.1
        outc_ref[:, 3:] = jnp.zeros((_BE, _CW - 3), F32)

    def wspec(r, c):
        return pl.BlockSpec((r, c), lambda i: (0, 0))

    return pl.pallas_call(
        body,
        grid=(_E // _BE,),
        in_specs=[
            pl.BlockSpec((_BE, _TW_G), lambda i: (i, 0)),
            pl.BlockSpec((_BE, _TW_G), lambda i: (i, 0)),
            wspec(_FEAT, _HID), wspec(_FEAT, _HID), wspec(1, _HID),
            wspec(1, _HID), wspec(_HID, _HID), wspec(1, _HID),
            wspec(_HID, 1), wspec(1, 1),
        ],
        out_specs=(
            pl.BlockSpec((_BE, _TW_E), lambda i: (i, 0)),
            pl.BlockSpec((_BE, _CW), lambda i: (i, 0)),
        ),
        out_shape=(
            jax.ShapeDtypeStruct((_E, _TW_E), BF16),
            jax.ShapeDtypeStruct((_E, _CW), F32),
        ),
    )(gs, gd, w1a, w1b, w1c, b1, w2, b2, w5, b5)


_BN = 2000


def _node_mlp(h2, pos2, agg0, agg1, aggc0, aggc1, w3a, w3b, b3, w4, b4):
    def body(h_ref, pos_ref, agg0_ref, agg1_ref, aggc0_ref, aggc1_ref,
             w3a_ref, w3b_ref, b3_ref, w4_ref, b4_ref, hn_ref, xn_ref):
        hh = h_ref[...]
        magg = agg0_ref[...].astype(F32) + agg1_ref[...].astype(F32)
        upd = aggc0_ref[:, :3] + aggc1_ref[:, :3]
        pre = (jnp.dot(hh, w3a_ref[...], preferred_element_type=F32)
               + jnp.dot(magg, w3b_ref[...], preferred_element_type=F32)
               + b3_ref[...])
        nh = pre * jax.nn.sigmoid(pre)
        hn_ref[...] = hh + jnp.dot(nh, w4_ref[...],
                                   preferred_element_type=F32) + b4_ref[...]
        xn_ref[...] = pos_ref[...] + upd

    def wspec(r, c):
        return pl.BlockSpec((r, c), lambda i: (0, 0))

    return pl.pallas_call(
        body,
        grid=(_N // _BN,),
        in_specs=[
            pl.BlockSpec((_BN, _FEAT), lambda i: (i, 0)),
            pl.BlockSpec((_BN, 3), lambda i: (i, 0)),
            pl.BlockSpec((_BN, _TW_E), lambda i: (i, 0)),
            pl.BlockSpec((_BN, _TW_E), lambda i: (i, 0)),
            pl.BlockSpec((_BN, _CW), lambda i: (i, 0)),
            pl.BlockSpec((_BN, _CW), lambda i: (i, 0)),
            wspec(_FEAT, _HID), wspec(_HID, _HID), wspec(1, _HID),
            wspec(_HID, _FEAT), wspec(1, _FEAT),
        ],
        out_specs=(
            pl.BlockSpec((_BN, _FEAT), lambda i: (i, 0)),
            pl.BlockSpec((_BN, 3), lambda i: (i, 0)),
        ),
        out_shape=(
            jax.ShapeDtypeStruct((_N, _FEAT), F32),
            jax.ShapeDtypeStruct((_N, 3), F32),
        ),
    )(h2, pos2, agg0, agg1, aggc0, aggc1, w3a, w3b, b3, w4, b4)


def kernel(h, pos, edge_index, W1, b1, W2, b2, W3, b3, W4, b4, W5, b5):
    h2 = h[0]
    pos2 = pos[0]
    src = edge_index[0]
    dst = edge_index[1]

    table = jnp.concatenate(
        [h2.astype(BF16), pos2.astype(BF16),
         jnp.zeros((_N, _TW_G - _FEAT - 3), BF16)], axis=1)
    gs, gd = _sc_gather(table, src, dst)

    eout, eoutc = _edge_mlp(
        gs, gd,
        W1[:_FEAT].astype(BF16), W1[_FEAT:2 * _FEAT].astype(BF16),
        W1[2 * _FEAT:2 * _FEAT + 1], b1[None, :], W2.astype(BF16),
        b2[None, :], W5.astype(BF16), b5[None, :])

    aggp, aggc = _sc_scatter(eout, eoutc, dst)

    h_new, x_new = _node_mlp(
        h2, pos2, aggp[0, :_N], aggp[1, :_N], aggc[0, :_N], aggc[1, :_N],
        W3[:_FEAT], W3[_FEAT:], b3[None, :], W4, b4[None, :])
    return h_new[None], x_new[None]
